# Initial kernel scaffold; baseline (speedup 1.0000x reference)
import jax
import jax.numpy as jnp
from jax.experimental import pallas as pl


def kernel(x, table):
    raise NotImplementedError("write your pallas kernel here")


# speedup vs baseline: 2.3200x; 2.3200x over previous
import functools

import jax
import jax.numpy as jnp
from jax import lax
from jax.experimental import pallas as pl
from jax.experimental.pallas import tpu as pltpu
from jax.experimental.pallas import tpu_sc as plsc


def _start_gather(tab_hbm, idx_ref, rows_v, sem):
    pltpu.make_async_copy(tab_hbm.at[idx_ref], rows_v, sem).start()


def _wait_gather(tab_hbm, idx_ref, rows_v, sem):
    pltpu.make_async_copy(tab_hbm.at[idx_ref], rows_v, sem).wait()


def _make_kernel(B, H, V, D):
    info = plsc.get_sparse_core_info()
    NC, NS, L = info.num_cores, info.num_subcores, info.num_lanes
    NW = NC * NS
    BPW = B // NW

    assert D == 2 * L
    UR = 8
    assert H % UR == 0
    NFULL = H // L
    TAIL = H - NFULL * L

    mesh = plsc.VectorSubcoreMesh(core_axis_name="c", subcore_axis_name="s")

    @functools.partial(
        pl.kernel,
        mesh=mesh,
        out_type=jax.ShapeDtypeStruct((B, D), jnp.float32),
        compiler_params=pltpu.CompilerParams(
            needs_layout_passes=False, use_tc_tiling_on_sc=False
        ),
        scratch_types=[
            pltpu.VMEM((BPW, H), jnp.int32),
            pltpu.VMEM((H, D), jnp.float32),
            pltpu.VMEM((H, D), jnp.float32),
            pltpu.VMEM((BPW, D), jnp.float32),
            pltpu.SemaphoreType.DMA,
            pltpu.SemaphoreType.DMA,
        ],
    )
    def k(x_hbm, tab_hbm, out_hbm, idx_v, rows0, rows1, out_v, sem0, sem1):
        wid = lax.axis_index("s") * NC + lax.axis_index("c")
        base = wid * BPW
        pltpu.sync_copy(x_hbm.at[pl.ds(base, BPW)], idx_v)

        lane = lax.iota(jnp.int32, L)
        zeros = jnp.full((L,), 0.0, jnp.float32)
        ones = jnp.full((L,), 1.0, jnp.float32)
        bufs = ((rows0, sem0), (rows1, sem1))

        _start_gather(tab_hbm, idx_v.at[0], rows0, sem0)
        _start_gather(tab_hbm, idx_v.at[1], rows1, sem1)

        def pair_body(i, carry):
            for bsel in range(2):
                rv, sem = bufs[bsel]
                r = 2 * i + bsel

                cntv = jnp.zeros((L,), jnp.int32)
                for c in range(NFULL):
                    ch = idx_v[r, pl.ds(c * L, L)]
                    cntv = cntv + plsc.all_reduce_population_count(ch != 0)
                if TAIL:
                    cht = idx_v[r, pl.ds(H - L, L)]
                    keep = (cht != 0) & (lane >= (L - TAIL))
                    cntv = cntv + plsc.all_reduce_population_count(keep)
                rec = ones / cntv.astype(jnp.float32)

                _wait_gather(tab_hbm, idx_v.at[r], rv, sem)

                def acc_body(j, acc):
                    a0, a1 = acc
                    o = j * UR
                    v0 = [rv[o + t, pl.ds(0, L)] for t in range(UR)]
                    v1 = [rv[o + t, pl.ds(L, L)] for t in range(UR)]

                    def tree(vs):
                        while len(vs) > 1:
                            nxt = [
                                vs[t] + vs[t + 1]
                                for t in range(0, len(vs) - 1, 2)
                            ]
                            if len(vs) % 2:
                                nxt.append(vs[-1])
                            vs = nxt
                        return vs[0]

                    return a0 + tree(v0), a1 + tree(v1)

                a0, a1 = lax.fori_loop(0, H // UR, acc_body, (zeros, zeros))

                nxt = r + 2

                @pl.when(nxt < BPW)
                def _():
                    _start_gather(tab_hbm, idx_v.at[nxt], rv, sem)

                out_v[r, pl.ds(0, L)] = a0 * rec
                out_v[r, pl.ds(L, L)] = a1 * rec
            return carry

        lax.fori_loop(0, BPW // 2, pair_body, 0)
        pltpu.sync_copy(out_v, out_hbm.at[pl.ds(base, BPW)])

    return k


def kernel(x, table):
    B, H = x.shape
    V, D = table.shape
    return _make_kernel(B, H, V, D)(x, table)
